# initial kernel scaffold (unmeasured)
import jax
import jax.numpy as jnp
from jax import lax
from jax.experimental import pallas as pl
from jax.experimental.pallas import tpu as pltpu


def kernel(
    x,
):
    def body(*refs):
        pass

    out_shape = jax.ShapeDtypeStruct(..., jnp.float32)
    return pl.pallas_call(body, out_shape=out_shape)(...)


# baseline (device time: 241916 ns/iter reference)
import jax
import jax.numpy as jnp
from jax import lax
from jax.experimental import pallas as pl
from jax.experimental.pallas import tpu as pltpu


def kernel(x):
    xb = x[0].astype(jnp.bfloat16)
    m, n2 = xb.shape
    n = n2 // 2

    def body(x_hbm, out_ref, recv_ref, send_sem, recv_sem, copy_sem):
        mx = lax.axis_index("x")
        my = lax.axis_index("y")
        peer_x = 1 - mx

        local = pltpu.make_async_copy(
            x_hbm.at[:, pl.ds(mx * n, n)], out_ref, copy_sem
        )
        local.start()

        rdma = pltpu.make_async_remote_copy(
            src_ref=x_hbm.at[:, pl.ds(peer_x * n, n)],
            dst_ref=recv_ref,
            send_sem=send_sem,
            recv_sem=recv_sem,
            device_id=(peer_x, my),
            device_id_type=pl.DeviceIdType.MESH,
        )
        rdma.start()
        local.wait()
        rdma.wait()
        out_ref[:, :] = out_ref[:, :] + recv_ref[:, :]

    return pl.pallas_call(
        body,
        out_shape=jax.ShapeDtypeStruct((m, n), jnp.bfloat16),
        in_specs=[pl.BlockSpec(memory_space=pltpu.HBM)],
        out_specs=pl.BlockSpec(memory_space=pltpu.VMEM),
        scratch_shapes=[
            pltpu.VMEM((m, n), jnp.bfloat16),
            pltpu.SemaphoreType.DMA,
            pltpu.SemaphoreType.DMA,
            pltpu.SemaphoreType.DMA,
        ],
        compiler_params=pltpu.CompilerParams(
            vmem_limit_bytes=56 * 1024 * 1024,
        ),
    )(xb)


# device time: 131791 ns/iter; 1.8356x vs baseline; 1.8356x over previous
import jax
import jax.numpy as jnp
from jax import lax
from jax.experimental import pallas as pl
from jax.experimental.pallas import tpu as pltpu

C = 16


def kernel(x):
    _, m_total, n2 = x.shape
    n = n2 // 2
    half_m = m_total // 2
    rc = half_m // C

    def body(x_hbm, out_ref, f32_buf, send_buf, recv_buf,
             copy_sems, xs_sems, xr_sems, ys_sems, yr_sems):
        mx = lax.axis_index("x")
        my = lax.axis_index("y")
        px = 1 - mx
        py = 1 - my
        row0 = my * half_m

        def local_dma(c):
            return pltpu.make_async_copy(
                x_hbm.at[0, pl.ds(row0 + c * rc, rc), :],
                f32_buf.at[c % 2],
                copy_sems.at[c % 2],
            )

        local_dma(0).start()
        x_rdmas = []
        for c in range(C):
            if c + 1 < C:
                local_dma(c + 1).start()
            local_dma(c).wait()
            rs = row0 + c * rc
            out_ref[pl.ds(rs, rc), :] = (
                f32_buf[c % 2, :, pl.ds(mx * n, n)].astype(jnp.bfloat16)
            )
            send_buf[c] = (
                f32_buf[c % 2, :, pl.ds(px * n, n)].astype(jnp.bfloat16)
            )
            rdma = pltpu.make_async_remote_copy(
                src_ref=send_buf.at[c],
                dst_ref=recv_buf.at[c],
                send_sem=xs_sems.at[c],
                recv_sem=xr_sems.at[c],
                device_id=(px, my),
                device_id_type=pl.DeviceIdType.MESH,
            )
            rdma.start()
            x_rdmas.append(rdma)

        y_rdmas = []
        for c in range(C):
            x_rdmas[c].wait_recv()
            rs = row0 + c * rc
            out_ref[pl.ds(rs, rc), :] = (
                out_ref[pl.ds(rs, rc), :] + recv_buf[c]
            )
            rdma = pltpu.make_async_remote_copy(
                src_ref=out_ref.at[pl.ds(rs, rc), :],
                dst_ref=out_ref.at[pl.ds(rs, rc), :],
                send_sem=ys_sems.at[c],
                recv_sem=yr_sems.at[c],
                device_id=(mx, py),
                device_id_type=pl.DeviceIdType.MESH,
            )
            rdma.start()
            y_rdmas.append(rdma)

        for c in range(C):
            y_rdmas[c].wait_recv()
        for c in range(C):
            x_rdmas[c].wait_send()
            y_rdmas[c].wait_send()

    return pl.pallas_call(
        body,
        out_shape=jax.ShapeDtypeStruct((m_total, n), jnp.bfloat16),
        in_specs=[pl.BlockSpec(memory_space=pltpu.HBM)],
        out_specs=pl.BlockSpec(memory_space=pltpu.VMEM),
        scratch_shapes=[
            pltpu.VMEM((2, rc, n2), jnp.float32),
            pltpu.VMEM((C, rc, n), jnp.bfloat16),
            pltpu.VMEM((C, rc, n), jnp.bfloat16),
            pltpu.SemaphoreType.DMA((2,)),
            pltpu.SemaphoreType.DMA((C,)),
            pltpu.SemaphoreType.DMA((C,)),
            pltpu.SemaphoreType.DMA((C,)),
            pltpu.SemaphoreType.DMA((C,)),
        ],
        compiler_params=pltpu.CompilerParams(
            vmem_limit_bytes=56 * 1024 * 1024,
        ),
    )(x)
